# Initial kernel scaffold; baseline (speedup 1.0000x reference)
import jax
import jax.numpy as jnp
from jax.experimental import pallas as pl


def kernel(x, edge_index, dt, norm, decay_lam):
    raise NotImplementedError("write your pallas kernel here")


# speedup vs baseline: 6.2304x; 6.2304x over previous
import functools

import jax
import jax.numpy as jnp
from jax import lax
from jax.experimental import pallas as pl
from jax.experimental.pallas import tpu as pltpu
from jax.experimental.pallas import tpu_sc as plsc

N_NODES = 10000
D = 128
E = 320000
NC = 2
NS = 16
NW = NC * NS
E_PER_W = E // NW
CHUNK = 80
N_CHUNKS = E_PER_W // CHUNK
WB_ROWS = 624
TAIL_ROWS = N_NODES - NS * WB_ROWS
ZROWS = 16
L = 16


def _sc_segment_sum(x, src3, dst3, dt2, norm2, lam16):
    mesh = plsc.VectorSubcoreMesh(core_axis_name="c", subcore_axis_name="s")

    @functools.partial(
        pl.kernel,
        out_type=jax.ShapeDtypeStruct((NC, N_NODES, D), jnp.float32),
        mesh=mesh,
        scratch_types=[
            pltpu.VMEM_SHARED((N_NODES, D), jnp.float32),
            pltpu.VMEM((N_CHUNKS, CHUNK), jnp.int32),
            pltpu.VMEM((N_CHUNKS, CHUNK), jnp.int32),
            pltpu.VMEM((CHUNK,), jnp.float32),
            pltpu.VMEM((CHUNK,), jnp.float32),
            pltpu.VMEM((CHUNK,), jnp.float32),
            pltpu.VMEM((L,), jnp.float32),
            pltpu.VMEM((CHUNK, D), jnp.float32),
            pltpu.VMEM((ZROWS, D), jnp.float32),
            pltpu.SemaphoreType.DMA,
        ],
    )
    def k(x_hbm, src_hbm, dst_hbm, dt_hbm, norm_hbm, lam_hbm, out_hbm,
          acc, srcv, dstv, dtc, normc, wc, lamv, rows, zbuf, sem):
        cid = lax.axis_index("c")
        sid = lax.axis_index("s")
        wid = sid * NC + cid

        pltpu.sync_copy(src_hbm.at[wid], srcv)
        pltpu.sync_copy(dst_hbm.at[wid], dstv)
        pltpu.sync_copy(lam_hbm, lamv)

        def zfill(i, _):
            for k2 in range(D // L):
                zbuf[i, pl.ds(k2 * L, L)] = jnp.zeros((L,), jnp.float32)
            return 0
        lax.fori_loop(0, ZROWS, zfill, 0)
        base_r = sid * WB_ROWS
        for t in range(WB_ROWS // ZROWS):
            pltpu.sync_copy(zbuf, acc.at[pl.ds(base_r + t * ZROWS, ZROWS)])
        @pl.when(sid == 0)
        def _zero_tail():
            pltpu.sync_copy(zbuf.at[pl.ds(0, TAIL_ROWS)],
                            acc.at[pl.ds(NS * WB_ROWS, TAIL_ROWS)])

        lamvec = lamv[...]

        plsc.subcore_barrier()

        def chunk_body(i, _):
            cp = pltpu.async_copy(x_hbm.at[srcv.at[i]], rows, sem)
            e0 = wid * E_PER_W + i * CHUNK
            pltpu.sync_copy(dt_hbm.at[pl.ds(e0, CHUNK)], dtc)
            pltpu.sync_copy(norm_hbm.at[pl.ds(e0, CHUNK)], normc)
            for j2 in range(CHUNK // L):
                sl2 = pl.ds(j2 * L, L)
                wc[sl2] = normc[sl2] * jnp.exp(-(lamvec * dtc[sl2]))
            cp.wait()

            def scale_body(j, _):
                wvec = wc[pl.ds(j * L, L)]
                for t in range(L):
                    e = j * L + t
                    ws = wvec[t]
                    for k2 in range(D // L):
                        sl = pl.ds(k2 * L, L)
                        rows[e, sl] = rows[e, sl] * ws
                return 0
            lax.fori_loop(0, CHUNK // L, scale_body, 0)

            pltpu.sync_copy(rows, acc.at[dstv.at[i]], add=True)
            return 0
        lax.fori_loop(0, N_CHUNKS, chunk_body, 0)

        plsc.subcore_barrier()

        pltpu.sync_copy(acc.at[pl.ds(base_r, WB_ROWS)],
                        out_hbm.at[cid, pl.ds(base_r, WB_ROWS)])
        @pl.when(sid == 0)
        def _write_tail():
            pltpu.sync_copy(acc.at[pl.ds(NS * WB_ROWS, TAIL_ROWS)],
                            out_hbm.at[cid, pl.ds(NS * WB_ROWS, TAIL_ROWS)])

    return k(x, src3, dst3, dt2, norm2, lam16)


def _combine(a, b):
    def body(a_ref, b_ref, o_ref):
        o_ref[...] = a_ref[...] + b_ref[...]
    return pl.pallas_call(
        body,
        out_shape=jax.ShapeDtypeStruct((N_NODES, D), jnp.float32),
    )(a, b)


def kernel(x, edge_index, dt, norm, decay_lam):
    src = edge_index[0].astype(jnp.int32).reshape(NW, N_CHUNKS, CHUNK)
    dst = edge_index[1].astype(jnp.int32).reshape(NW, N_CHUNKS, CHUNK)
    dt2 = dt.astype(jnp.float32)
    norm2 = norm.astype(jnp.float32)
    lam = jnp.maximum(decay_lam.astype(jnp.float32), 0.0) + 0.0001
    lam16 = jnp.full((L,), lam, jnp.float32)
    parts = _sc_segment_sum(x, src, dst, dt2, norm2, lam16)
    return _combine(parts[0], parts[1])
